# baseline (speedup 1.0000x reference)
import functools

import jax
import jax.numpy as jnp
from jax import lax
from jax.experimental import pallas as pl
from jax.experimental.pallas import tpu as pltpu
from jax.experimental.pallas import tpu_sc as plsc

SEQ = 16384
Q = 32
D = 4 * Q
NC = 2
NS = 16
NW = NC * NS
BPW = SEQ // NW
L = 16
G = BPW // L

_TAB_ROWS = (24, 7, 12, 10)

_mesh = plsc.VectorSubcoreMesh(core_axis_name="c", subcore_axis_name="s")


@functools.partial(
    pl.kernel,
    out_type=jax.ShapeDtypeStruct((SEQ, D), jnp.float32),
    mesh=_mesh,
    compiler_params=pltpu.CompilerParams(needs_layout_passes=False),
    scratch_types=[
        pltpu.VMEM((BPW,), jnp.int32),
        pltpu.VMEM((BPW,), jnp.int32),
        pltpu.VMEM((BPW,), jnp.int32),
        pltpu.VMEM((BPW,), jnp.int32),
        pltpu.VMEM((_TAB_ROWS[0] * Q,), jnp.float32),
        pltpu.VMEM((_TAB_ROWS[1] * Q,), jnp.float32),
        pltpu.VMEM((_TAB_ROWS[2] * Q,), jnp.float32),
        pltpu.VMEM((_TAB_ROWS[3] * Q,), jnp.float32),
        pltpu.VMEM((BPW, D), jnp.float32),
    ],
)
def _encode(hours, days, months, years, wh, wd, wm, wy, out,
            ih_v, id_v, im_v, iy_v, th_v, td_v, tm_v, ty_v, out_v):
    wid = lax.axis_index("s") * NC + lax.axis_index("c")
    base = wid * BPW

    for src, dst in ((wh, th_v), (wd, td_v), (wm, tm_v), (wy, ty_v)):
        pltpu.sync_copy(src, dst)
    idx_vs = (ih_v, id_v, im_v, iy_v)
    for src, dst in zip((hours, days, months, years), idx_vs):
        pltpu.sync_copy(src.at[pl.ds(base, BPW)], dst)

    lanes = lax.iota(jnp.int32, L)
    tab_vs = (th_v, td_v, tm_v, ty_v)

    @plsc.parallel_loop(0, G, unroll=4)
    def body(g):
        row0 = g * L
        rows = row0 + lanes
        for t in range(4):
            v = idx_vs[t][pl.ds(row0, L)]
            src0 = v * Q
            for c in range(Q):
                w = plsc.load_gather(tab_vs[t], [src0 + c])
                plsc.store_scatter(out_v, [rows, jnp.full((L,), t * Q + c,
                                                          jnp.int32)], w)

    pltpu.sync_copy(out_v, out.at[pl.ds(base, BPW), :])


def kernel(hours, days, months, years, W_hour, W_day, W_month, W_year):
    return _encode(hours.astype(jnp.int32), days.astype(jnp.int32),
                   months.astype(jnp.int32), years.astype(jnp.int32),
                   W_hour.reshape(-1), W_day.reshape(-1),
                   W_month.reshape(-1), W_year.reshape(-1))


# speedup vs baseline: 1.0068x; 1.0068x over previous
import functools

import jax
import jax.numpy as jnp
from jax import lax
from jax.experimental import pallas as pl
from jax.experimental.pallas import tpu as pltpu
from jax.experimental.pallas import tpu_sc as plsc

SEQ = 16384
Q = 32
D = 4 * Q
NC = 2
NS = 16
NW = NC * NS
BPW = SEQ // NW
L = 16
G = BPW // L

_TAB_ROWS = (24, 7, 12, 10)

_mesh = plsc.VectorSubcoreMesh(core_axis_name="c", subcore_axis_name="s")


@functools.partial(
    pl.kernel,
    out_type=jax.ShapeDtypeStruct((SEQ, D), jnp.float32),
    mesh=_mesh,
    compiler_params=pltpu.CompilerParams(needs_layout_passes=False),
    scratch_types=[
        pltpu.VMEM((BPW,), jnp.int32),
        pltpu.VMEM((BPW,), jnp.int32),
        pltpu.VMEM((BPW,), jnp.int32),
        pltpu.VMEM((BPW,), jnp.int32),
        pltpu.VMEM((_TAB_ROWS[0] * Q,), jnp.float32),
        pltpu.VMEM((_TAB_ROWS[1] * Q,), jnp.float32),
        pltpu.VMEM((_TAB_ROWS[2] * Q,), jnp.float32),
        pltpu.VMEM((_TAB_ROWS[3] * Q,), jnp.float32),
        pltpu.VMEM((BPW, D), jnp.float32),
    ],
)
def _encode(hours, days, months, years, wh, wd, wm, wy, out,
            ih_v, id_v, im_v, iy_v, th_v, td_v, tm_v, ty_v, out_v):
    wid = lax.axis_index("s") * NC + lax.axis_index("c")
    base = wid * BPW

    for src, dst in ((wh, th_v), (wd, td_v), (wm, tm_v), (wy, ty_v)):
        pltpu.sync_copy(src, dst)
    idx_vs = (ih_v, id_v, im_v, iy_v)
    for src, dst in zip((hours, days, months, years), idx_vs):
        pltpu.sync_copy(src.at[pl.ds(base, BPW)], dst)

    lanes = lax.iota(jnp.int32, L)
    tab_vs = (th_v, td_v, tm_v, ty_v)

    @plsc.parallel_loop(0, G, unroll=2)
    def body(g):
        row0 = g * L
        rows = row0 + lanes
        for t in range(4):
            v = idx_vs[t][pl.ds(row0, L)]
            src0 = v * Q
            for c in range(Q):
                w = plsc.load_gather(tab_vs[t], [src0 + c])
                plsc.store_scatter(out_v, [rows, jnp.full((L,), t * Q + c,
                                                          jnp.int32)], w)

    pltpu.sync_copy(out_v, out.at[pl.ds(base, BPW), :])


def kernel(hours, days, months, years, W_hour, W_day, W_month, W_year):
    return _encode(hours.astype(jnp.int32), days.astype(jnp.int32),
                   months.astype(jnp.int32), years.astype(jnp.int32),
                   W_hour.reshape(-1), W_day.reshape(-1),
                   W_month.reshape(-1), W_year.reshape(-1))


# speedup vs baseline: 1.2812x; 1.2726x over previous
import functools

import jax
import jax.numpy as jnp
from jax import lax
from jax.experimental import pallas as pl
from jax.experimental.pallas import tpu as pltpu
from jax.experimental.pallas import tpu_sc as plsc

SEQ = 16384
Q = 32
D = 4 * Q
NC = 2
NS = 16
NW = NC * NS
BPW = SEQ // NW
L = 16
G = BPW // L

_TAB_ROWS = (24, 7, 12, 10)

_mesh = plsc.VectorSubcoreMesh(core_axis_name="c", subcore_axis_name="s")


@functools.partial(
    pl.kernel,
    out_type=jax.ShapeDtypeStruct((SEQ, D), jnp.float32),
    mesh=_mesh,
    compiler_params=pltpu.CompilerParams(needs_layout_passes=False),
    scratch_types=[
        pltpu.VMEM((BPW,), jnp.int32),
        pltpu.VMEM((BPW,), jnp.int32),
        pltpu.VMEM((BPW,), jnp.int32),
        pltpu.VMEM((BPW,), jnp.int32),
        pltpu.VMEM((_TAB_ROWS[0] * (Q + 1),), jnp.float32),
        pltpu.VMEM((_TAB_ROWS[1] * (Q + 1),), jnp.float32),
        pltpu.VMEM((_TAB_ROWS[2] * (Q + 1),), jnp.float32),
        pltpu.VMEM((_TAB_ROWS[3] * (Q + 1),), jnp.float32),
        pltpu.VMEM((BPW, D), jnp.float32),
    ],
)
def _encode(hours, days, months, years, wh, wd, wm, wy, out,
            ih_v, id_v, im_v, iy_v, th_v, td_v, tm_v, ty_v, out_v):
    wid = lax.axis_index("s") * NC + lax.axis_index("c")
    base = wid * BPW

    for src, dst in ((wh, th_v), (wd, td_v), (wm, tm_v), (wy, ty_v)):
        pltpu.sync_copy(src, dst)
    idx_vs = (ih_v, id_v, im_v, iy_v)
    for src, dst in zip((hours, days, months, years), idx_vs):
        pltpu.sync_copy(src.at[pl.ds(base, BPW)], dst)

    lanes = lax.iota(jnp.int32, L)
    tab_vs = (th_v, td_v, tm_v, ty_v)

    @plsc.parallel_loop(0, G, unroll=2)
    def body(g):
        row0 = g * L
        rows = row0 + lanes
        for t in range(4):
            v = idx_vs[t][pl.ds(row0, L)]
            src0 = v * (Q + 1)
            for c in range(Q):
                w = plsc.load_gather(tab_vs[t], [src0 + c])
                plsc.store_scatter(out_v, [rows, jnp.full((L,), t * Q + c,
                                                          jnp.int32)], w)

    pltpu.sync_copy(out_v, out.at[pl.ds(base, BPW), :])


def kernel(hours, days, months, years, W_hour, W_day, W_month, W_year):
    pad = ((0, 0), (0, 1))
    return _encode(hours.astype(jnp.int32), days.astype(jnp.int32),
                   months.astype(jnp.int32), years.astype(jnp.int32),
                   jnp.pad(W_hour, pad).reshape(-1),
                   jnp.pad(W_day, pad).reshape(-1),
                   jnp.pad(W_month, pad).reshape(-1),
                   jnp.pad(W_year, pad).reshape(-1))


# speedup vs baseline: 2.3969x; 1.8708x over previous
import functools

import jax
import jax.numpy as jnp
from jax import lax
from jax.experimental import pallas as pl
from jax.experimental.pallas import tpu as pltpu
from jax.experimental.pallas import tpu_sc as plsc

SEQ = 16384
Q = 32
D = 4 * Q
NC = 2
NS = 16
NW = NC * NS
BPW = SEQ // NW
L = 16

_TAB_ROWS = (24, 7, 12, 10)

_mesh = plsc.VectorSubcoreMesh(core_axis_name="c", subcore_axis_name="s")


@functools.partial(
    pl.kernel,
    out_type=jax.ShapeDtypeStruct((SEQ * D,), jnp.float32),
    mesh=_mesh,
    compiler_params=pltpu.CompilerParams(needs_layout_passes=False),
    scratch_types=[
        pltpu.VMEM((BPW,), jnp.int32),
        pltpu.VMEM((BPW,), jnp.int32),
        pltpu.VMEM((BPW,), jnp.int32),
        pltpu.VMEM((BPW,), jnp.int32),
        pltpu.VMEM((_TAB_ROWS[0] * Q,), jnp.float32),
        pltpu.VMEM((_TAB_ROWS[1] * Q,), jnp.float32),
        pltpu.VMEM((_TAB_ROWS[2] * Q,), jnp.float32),
        pltpu.VMEM((_TAB_ROWS[3] * Q,), jnp.float32),
        pltpu.VMEM((BPW * D,), jnp.float32),
    ],
)
def _encode(hours, days, months, years, wh, wd, wm, wy, out,
            ih_v, id_v, im_v, iy_v, th_v, td_v, tm_v, ty_v, out_v):
    wid = lax.axis_index("s") * NC + lax.axis_index("c")
    base = wid * BPW

    for src, dst in ((wh, th_v), (wd, td_v), (wm, tm_v), (wy, ty_v)):
        pltpu.sync_copy(src, dst)
    idx_vs = (ih_v, id_v, im_v, iy_v)
    for src, dst in zip((hours, days, months, years), idx_vs):
        pltpu.sync_copy(src.at[pl.ds(base, BPW)], dst)

    tab_vs = (th_v, td_v, tm_v, ty_v)

    @plsc.parallel_loop(0, BPW // L, unroll=2)
    def body(g):
        row0 = g * L
        vqs = [idx_vs[t][pl.ds(row0, L)] * Q for t in range(4)]
        for j in range(L):
            dst0 = (row0 + j) * D
            for t in range(4):
                s = vqs[t][j]
                for c0 in range(0, Q, L):
                    w = tab_vs[t][pl.ds(s + c0, L)]
                    out_v[pl.ds(dst0 + t * Q + c0, L)] = w

    pltpu.sync_copy(out_v, out.at[pl.ds(base * D, BPW * D)])


def kernel(hours, days, months, years, W_hour, W_day, W_month, W_year):
    flat = _encode(hours.astype(jnp.int32), days.astype(jnp.int32),
                   months.astype(jnp.int32), years.astype(jnp.int32),
                   W_hour.reshape(-1), W_day.reshape(-1),
                   W_month.reshape(-1), W_year.reshape(-1))
    return flat.reshape(SEQ, D)
